# baseline (speedup 1.0000x reference)
import jax
import jax.numpy as jnp
from jax.experimental import pallas as pl
from jax.experimental.pallas import tpu as pltpu

B = 128
D_IN = 2048
D_HID = 16384
K_SEL = 64
H_BLK = 1024
N_HBLK = D_HID // H_BLK
C_BLK = 2048
N_CBLK = D_HID // C_BLK


def _i16(x32):
    return (x32 - 32768).astype(jnp.int16)


def _count_mask16(mask):
    m = jnp.where(mask, jnp.int16(1), jnp.int16(0))
    w = m.shape[1]
    while w > 128:
        half = w // 2
        m = m[:, :half] + m[:, half:w]
        w = half
    return jnp.sum(m.astype(jnp.int32), axis=1, keepdims=True)


def _keymap(v):
    bits = jax.lax.bitcast_convert_type(v, jnp.int32)
    key = jnp.where(bits >= 0, bits, bits ^ jnp.int32(0x7FFFFFFF))
    return jax.lax.bitcast_convert_type(key, jnp.uint32) ^ jnp.uint32(0x80000000)


def _hi_lo(v):
    ub = _keymap(v)
    hi = jax.lax.shift_right_logical(ub, jnp.uint32(16)).astype(jnp.int32)
    lo = (ub & jnp.uint32(0xFFFF)).astype(jnp.int32)
    return hi, lo


_PRE_LEVELS = [[(2 * m + 1) << (15 - l) for m in range(1 << l)]
               for l in range(3)]
_PRE_THRESH = [v for lv in _PRE_LEVELS for v in lv]


def _find_thresholds(hr_ref, s1_ref, s2_ref, s3_ref, t1_ref, t2_ref, t3_ref,
                     pc_ref):
    pre = {v: pc_ref[i][:, 0:1] for i, v in enumerate(_PRE_THRESH)}
    t = jnp.zeros((B, 1), jnp.int32)
    for l, level in enumerate(_PRE_LEVELS):
        cand = t | (1 << (15 - l))
        cnt = jnp.zeros((B, 1), jnp.int32)
        for v in level:
            cnt = jnp.where(cand == v, pre[v], cnt)
        t = jnp.where(cnt >= K_SEL, cand, t)
    for bit in range(12, -1, -1):
        cnt = _count_mask16(s1_ref[...] >= _i16(t | (1 << bit)))
        t = jnp.where(cnt >= K_SEL, t | (1 << bit), t)
    t_s = _i16(t)
    cnt_gt1 = _count_mask16(s1_ref[...] > t_s)
    need1 = K_SEL - cnt_gt1
    for c in range(N_CBLK):
        sl = pl.ds(c * C_BLK, C_BLK)
        _, lo = _hi_lo(hr_ref[:, sl])
        s2_ref[:, sl] = jnp.where(s1_ref[:, sl] == t_s, _i16(lo),
                                  jnp.int16(-32768))
    tl = jnp.zeros((B, 1), jnp.int32)
    cnt_at_tl = jnp.full((B, 1), D_HID, jnp.int32)
    for bit in range(15, -1, -1):
        cnt = _count_mask16(s2_ref[...] >= _i16(tl | (1 << bit)))
        acc = cnt >= need1
        tl = jnp.where(acc, tl | (1 << bit), tl)
        cnt_at_tl = jnp.where(acc, cnt, cnt_at_tl)
    tl_s = _i16(tl)
    t1_ref[...] = jnp.broadcast_to(t_s, (B, 128))
    t2_ref[...] = jnp.broadcast_to(tl_s, (B, 128))
    ties_excess = jnp.any(cnt_at_tl > need1)

    @pl.when(jnp.logical_not(ties_excess))
    def _fast():
        t3_ref[...] = jnp.full((B, 128), -32768, jnp.int16)

    @pl.when(ties_excess)
    def _slow():
        cnt_gt2 = _count_mask16(s2_ref[...] > tl_s)
        need2 = need1 - cnt_gt2
        for c in range(N_CBLK):
            sl = pl.ds(c * C_BLK, C_BLK)
            ridx16 = ((D_HID - 1 - c * C_BLK)
                      - jax.lax.broadcasted_iota(jnp.int32, (B, C_BLK), 1)
                      ).astype(jnp.int16)
            eq_c = jnp.logical_and(s1_ref[:, sl] == t_s,
                                   s2_ref[:, sl] == tl_s)
            s3_ref[:, sl] = jnp.where(eq_c, ridx16, jnp.int16(-1))
        r = jnp.zeros((B, 1), jnp.int32)
        for bit in range(13, -1, -1):
            cnt = _count_mask16(s3_ref[...] >= (r | (1 << bit)).astype(jnp.int16))
            r = jnp.where(cnt >= need2, r | (1 << bit), r)
        t3_ref[...] = jnp.broadcast_to(r.astype(jnp.int16), (B, 128))


def _fused_kernel(x_ref, we_ref, be_ref, wd_ref, bd_ref, h_ref, o_ref,
                  hr_ref, s1_ref, s2_ref, s3_ref, t1_ref, t2_ref, t3_ref,
                  pc_ref):
    j = pl.program_id(0)

    @pl.when(j < N_HBLK)
    def _encode():
        blk = jax.lax.dot_general(x_ref[...], we_ref[...],
                                  (((1,), (1,)), ((), ())),
                                  preferred_element_type=jnp.float32)
        blk = blk + be_ref[...]
        sl = pl.ds(j * H_BLK, H_BLK)
        hr_ref[:, sl] = blk
        hi, _ = _hi_lo(blk)
        s1c = _i16(hi)
        s1_ref[:, sl] = s1c
        for i, v in enumerate(_PRE_THRESH):
            cnt = jnp.broadcast_to(_count_mask16(s1c >= jnp.int16(v - 32768)),
                                   (B, 128))
            prev = jnp.where(j == 0, jnp.int32(0), pc_ref[i])
            pc_ref[i] = prev + cnt

        @pl.when(j == N_HBLK - 1)
        def _():
            _find_thresholds(hr_ref, s1_ref, s2_ref, s3_ref,
                             t1_ref, t2_ref, t3_ref, pc_ref)

    @pl.when(j >= N_HBLK)
    def _decode():
        jj = j - N_HBLK

        @pl.when(jj == 0)
        def _():
            o_ref[...] = jnp.broadcast_to(bd_ref[...], (B, D_IN))

        sl = pl.ds(jj * H_BLK, H_BLK)
        t_s = t1_ref[...][:, 0:1]
        tl_s = t2_ref[...][:, 0:1]
        r_s = t3_ref[...][:, 0:1]
        s1c = s1_ref[:, sl]
        s2c = s2_ref[:, sl]
        eq_hi = s1c == t_s
        keep = jnp.logical_or(s1c > t_s,
                              jnp.logical_and(eq_hi, s2c > tl_s))
        keep = jnp.logical_or(
            keep,
            jnp.logical_and(jnp.logical_and(eq_hi, s2c == tl_s),
                            s3_ref[:, sl] >= r_s))
        h_c = jnp.where(keep, hr_ref[:, sl], jnp.float32(0.0))
        h_ref[...] = h_c
        o_ref[...] += jax.lax.dot_general(h_c, wd_ref[...],
                                          (((1,), (1,)), ((), ())),
                                          preferred_element_type=jnp.float32)


def kernel(x, W_enc, b_enc, W_dec, b_dec):
    b_enc2 = b_enc.reshape(1, D_HID)
    b_dec2 = b_dec.reshape(1, D_IN)
    n = N_HBLK

    h, x_hat = pl.pallas_call(
        _fused_kernel,
        grid=(2 * n,),
        in_specs=[
            pl.BlockSpec((B, D_IN), lambda j: (0, 0)),
            pl.BlockSpec((H_BLK, D_IN), lambda j: (jnp.minimum(j, n - 1), 0)),
            pl.BlockSpec((1, H_BLK), lambda j: (0, jnp.minimum(j, n - 1))),
            pl.BlockSpec((D_IN, H_BLK), lambda j: (0, jnp.maximum(j - n, 0))),
            pl.BlockSpec((1, D_IN), lambda j: (0, 0)),
        ],
        out_specs=[
            pl.BlockSpec((B, H_BLK), lambda j: (0, jnp.maximum(j - n, 0))),
            pl.BlockSpec((B, D_IN), lambda j: (0, 0)),
        ],
        out_shape=[
            jax.ShapeDtypeStruct((B, D_HID), jnp.float32),
            jax.ShapeDtypeStruct((B, D_IN), jnp.float32),
        ],
        scratch_shapes=[
            pltpu.VMEM((B, D_HID), jnp.float32),
            pltpu.VMEM((B, D_HID), jnp.int16),
            pltpu.VMEM((B, D_HID), jnp.int16),
            pltpu.VMEM((B, D_HID), jnp.int16),
            pltpu.VMEM((B, 128), jnp.int16),
            pltpu.VMEM((B, 128), jnp.int16),
            pltpu.VMEM((B, 128), jnp.int16),
            pltpu.VMEM((7, B, 128), jnp.int32),
        ],
    )(x, W_enc, b_enc2, W_dec, b_dec2)

    return (h, x_hat)


# speedup vs baseline: 1.0191x; 1.0191x over previous
import jax
import jax.numpy as jnp
from jax.experimental import pallas as pl
from jax.experimental.pallas import tpu as pltpu

B = 128
D_IN = 2048
D_HID = 16384
K_SEL = 64
H_BLK = 1024
N_HBLK = D_HID // H_BLK
C_BLK = 2048
N_CBLK = D_HID // C_BLK


def _i16(x32):
    return (x32 - 32768).astype(jnp.int16)


def _count_mask16(mask):
    m = jnp.where(mask, jnp.int16(1), jnp.int16(0))
    w = m.shape[1]
    while w > 128:
        half = w // 2
        m = m[:, :half] + m[:, half:w]
        w = half
    return jnp.sum(m.astype(jnp.int32), axis=1, keepdims=True)


def _keymap(v):
    bits = jax.lax.bitcast_convert_type(v, jnp.int32)
    key = jnp.where(bits >= 0, bits, bits ^ jnp.int32(0x7FFFFFFF))
    return jax.lax.bitcast_convert_type(key, jnp.uint32) ^ jnp.uint32(0x80000000)


def _hi_lo(v):
    ub = _keymap(v)
    hi = jax.lax.shift_right_logical(ub, jnp.uint32(16)).astype(jnp.int32)
    lo = (ub & jnp.uint32(0xFFFF)).astype(jnp.int32)
    return hi, lo


def _find_thresholds(hr_ref, s1_ref, s2_ref, s3_ref, t1_ref, t2_ref, t3_ref):
    t = jnp.zeros((B, 1), jnp.int32)
    for bit in range(15, -1, -1):
        cnt = _count_mask16(s1_ref[...] >= _i16(t | (1 << bit)))
        t = jnp.where(cnt >= K_SEL, t | (1 << bit), t)
    t_s = _i16(t)
    cnt_gt1 = _count_mask16(s1_ref[...] > t_s)
    need1 = K_SEL - cnt_gt1
    for c in range(N_CBLK):
        sl = pl.ds(c * C_BLK, C_BLK)
        _, lo = _hi_lo(hr_ref[:, sl])
        s2_ref[:, sl] = jnp.where(s1_ref[:, sl] == t_s, _i16(lo),
                                  jnp.int16(-32768))
    tl = jnp.zeros((B, 1), jnp.int32)
    cnt_at_tl = jnp.full((B, 1), D_HID, jnp.int32)
    for bit in range(15, -1, -1):
        cnt = _count_mask16(s2_ref[...] >= _i16(tl | (1 << bit)))
        acc = cnt >= need1
        tl = jnp.where(acc, tl | (1 << bit), tl)
        cnt_at_tl = jnp.where(acc, cnt, cnt_at_tl)
    tl_s = _i16(tl)
    t1_ref[...] = jnp.broadcast_to(t_s, (B, 128))
    t2_ref[...] = jnp.broadcast_to(tl_s, (B, 128))
    ties_excess = jnp.any(cnt_at_tl > need1)

    @pl.when(jnp.logical_not(ties_excess))
    def _fast():
        t3_ref[...] = jnp.full((B, 128), -32768, jnp.int16)

    @pl.when(ties_excess)
    def _slow():
        cnt_gt2 = _count_mask16(s2_ref[...] > tl_s)
        need2 = need1 - cnt_gt2
        for c in range(N_CBLK):
            sl = pl.ds(c * C_BLK, C_BLK)
            ridx16 = ((D_HID - 1 - c * C_BLK)
                      - jax.lax.broadcasted_iota(jnp.int32, (B, C_BLK), 1)
                      ).astype(jnp.int16)
            eq_c = jnp.logical_and(s1_ref[:, sl] == t_s,
                                   s2_ref[:, sl] == tl_s)
            s3_ref[:, sl] = jnp.where(eq_c, ridx16, jnp.int16(-1))
        r = jnp.zeros((B, 1), jnp.int32)
        for bit in range(13, -1, -1):
            cnt = _count_mask16(s3_ref[...] >= (r | (1 << bit)).astype(jnp.int16))
            r = jnp.where(cnt >= need2, r | (1 << bit), r)
        t3_ref[...] = jnp.broadcast_to(r.astype(jnp.int16), (B, 128))


def _fused_kernel(x_ref, we_ref, be_ref, wd_ref, bd_ref, h_ref, o_ref,
                  hr_ref, s1_ref, s2_ref, s3_ref, t1_ref, t2_ref, t3_ref):
    j = pl.program_id(0)

    @pl.when(j < N_HBLK)
    def _encode():
        blk = jax.lax.dot_general(x_ref[...], we_ref[...],
                                  (((1,), (1,)), ((), ())),
                                  preferred_element_type=jnp.float32)
        blk = blk + be_ref[...]
        sl = pl.ds(j * H_BLK, H_BLK)
        hr_ref[:, sl] = blk
        hi, _ = _hi_lo(blk)
        s1_ref[:, sl] = _i16(hi)

        @pl.when(j == N_HBLK - 1)
        def _():
            _find_thresholds(hr_ref, s1_ref, s2_ref, s3_ref,
                             t1_ref, t2_ref, t3_ref)

    @pl.when(j >= N_HBLK)
    def _decode():
        jj = j - N_HBLK

        @pl.when(jj == 0)
        def _():
            o_ref[...] = jnp.broadcast_to(bd_ref[...], (B, D_IN))

        sl = pl.ds(jj * H_BLK, H_BLK)
        t_s = t1_ref[...][:, 0:1]
        tl_s = t2_ref[...][:, 0:1]
        r_s = t3_ref[...][:, 0:1]
        s1c = s1_ref[:, sl]
        s2c = s2_ref[:, sl]
        eq_hi = s1c == t_s
        keep = jnp.logical_or(s1c > t_s,
                              jnp.logical_and(eq_hi, s2c > tl_s))
        keep = jnp.logical_or(
            keep,
            jnp.logical_and(jnp.logical_and(eq_hi, s2c == tl_s),
                            s3_ref[:, sl] >= r_s))
        h_c = jnp.where(keep, hr_ref[:, sl], jnp.float32(0.0))
        h_ref[...] = h_c
        o_ref[...] += jax.lax.dot_general(h_c, wd_ref[...],
                                          (((1,), (1,)), ((), ())),
                                          preferred_element_type=jnp.float32)


def kernel(x, W_enc, b_enc, W_dec, b_dec):
    b_enc2 = b_enc.reshape(1, D_HID)
    b_dec2 = b_dec.reshape(1, D_IN)
    n = N_HBLK

    h, x_hat = pl.pallas_call(
        _fused_kernel,
        grid=(2 * n,),
        in_specs=[
            pl.BlockSpec((B, D_IN), lambda j: (0, 0)),
            pl.BlockSpec((H_BLK, D_IN), lambda j: (jnp.minimum(j, n - 1), 0)),
            pl.BlockSpec((1, H_BLK), lambda j: (0, jnp.minimum(j, n - 1))),
            pl.BlockSpec((D_IN, H_BLK), lambda j: (0, jnp.maximum(j - n, 0))),
            pl.BlockSpec((1, D_IN), lambda j: (0, 0)),
        ],
        out_specs=[
            pl.BlockSpec((B, H_BLK), lambda j: (0, jnp.maximum(j - n, 0))),
            pl.BlockSpec((B, D_IN), lambda j: (0, 0)),
        ],
        out_shape=[
            jax.ShapeDtypeStruct((B, D_HID), jnp.float32),
            jax.ShapeDtypeStruct((B, D_IN), jnp.float32),
        ],
        scratch_shapes=[
            pltpu.VMEM((B, D_HID), jnp.float32),
            pltpu.VMEM((B, D_HID), jnp.int16),
            pltpu.VMEM((B, D_HID), jnp.int16),
            pltpu.VMEM((B, D_HID), jnp.int16),
            pltpu.VMEM((B, 128), jnp.int16),
            pltpu.VMEM((B, 128), jnp.int16),
            pltpu.VMEM((B, 128), jnp.int16),
        ],
    )(x, W_enc, b_enc2, W_dec, b_dec2)

    return (h, x_hat)
